# baseline (speedup 1.0000x reference)
import functools

import jax
import jax.numpy as jnp
from jax.experimental import pallas as pl
from jax.experimental.pallas import tpu as pltpu


def _h0_body(xt_ref, u0t_ref, b0_ref, h0t_ref, *, n, bj):
    j = pl.program_id(0)
    h0t = jax.lax.dot_general(u0t_ref[...], xt_ref[...],
                              (((1,), (0,)), ((), ())),
                              preferred_element_type=jnp.float32)
    h0t = h0t + b0_ref[...]
    col = jax.lax.broadcasted_iota(jnp.int32, h0t.shape, 1)
    h0t_ref[...] = jnp.where(col < n - j * bj, h0t, 0.0).astype(jnp.bfloat16)


def _stream_layer(ht_ref, a_ref, acc_ref):
    j = pl.program_id(1)
    a_blk = a_ref[...].astype(jnp.bfloat16)
    part = jax.lax.dot_general(ht_ref[...], a_blk, (((1,), (0,)), ((), ())),
                               preferred_element_type=jnp.float32)

    @pl.when(j == 0)
    def _():
        acc_ref[...] = part

    @pl.when(j != 0)
    def _():
        acc_ref[...] = acc_ref[...] + part

    return a_blk


def _layer0_body(ht_ref, a_ref, u_ref, h1t_ref, deg_out_ref,
                 acc_ref, deg_ref, *, n, bi, bj):
    i = pl.program_id(0)
    j = pl.program_id(1)
    nj = pl.num_programs(1)
    a_blk = _stream_layer(ht_ref, a_ref, acc_ref)

    row = jax.lax.broadcasted_iota(jnp.int32, (1, bj), 1)
    valid = (row < n - j * bj).astype(jnp.bfloat16)
    degp = jax.lax.dot_general(valid, a_blk, (((1,), (0,)), ((), ())),
                               preferred_element_type=jnp.float32)

    @pl.when(j == 0)
    def _():
        deg_ref[...] = degp

    @pl.when(j != 0)
    def _():
        deg_ref[...] = deg_ref[...] + degp

    @pl.when(j == nj - 1)
    def _():
        deg_out_ref[...] = deg_ref[...]
        scaled = (acc_ref[...] / deg_ref[...]).astype(jnp.bfloat16)
        y = jax.lax.dot_general(u_ref[...], scaled, (((1,), (0,)), ((), ())),
                                preferred_element_type=jnp.float32)
        y = jnp.maximum(y, 0.0)
        col = jax.lax.broadcasted_iota(jnp.int32, y.shape, 1)
        h1t_ref[...] = jnp.where(col < n - i * bi, y, 0.0).astype(jnp.bfloat16)


def _layer1_body(ht_ref, a_ref, u_ref, deg_ref, out_ref, acc_ref):
    j = pl.program_id(1)
    nj = pl.num_programs(1)
    _stream_layer(ht_ref, a_ref, acc_ref)

    @pl.when(j == nj - 1)
    def _():
        scaled = (acc_ref[...] / deg_ref[...]).astype(jnp.bfloat16)
        y = jax.lax.dot_general(u_ref[...], scaled, (((1,), (0,)), ((), ())),
                                preferred_element_type=jnp.float32)
        y = jnp.maximum(y, 0.0)
        out_ref[...] = jnp.swapaxes(y, 0, 1)


def kernel(x, adj_mat, U0, b0, U1, U2):
    n, d = x.shape
    bi = 1280 if n > 1280 else n // 8
    bj = 2560 if n > 1280 else n // 5
    ni = -(-n // bi)
    nj = -(-n // bj)
    n_pad = max(ni * bi, nj * bj)

    xt = jnp.zeros((d, n_pad), jnp.bfloat16).at[:, :n].set(
        x.astype(jnp.bfloat16).T)
    u0t = U0.T.astype(jnp.bfloat16)
    u1 = U1.astype(jnp.bfloat16)
    u2 = U2.astype(jnp.bfloat16)
    b0c = b0.reshape(d, 1)

    h0t = pl.pallas_call(
        functools.partial(_h0_body, n=n, bj=bj),
        grid=(nj,),
        in_specs=[
            pl.BlockSpec((d, bj), lambda j: (0, j)),
            pl.BlockSpec((d, d), lambda j: (0, 0)),
            pl.BlockSpec((d, 1), lambda j: (0, 0)),
        ],
        out_specs=pl.BlockSpec((d, bj), lambda j: (0, j)),
        out_shape=jax.ShapeDtypeStruct((d, n_pad), jnp.bfloat16),
    )(xt, u0t, b0c)

    h1t, deg = pl.pallas_call(
        functools.partial(_layer0_body, n=n, bi=bi, bj=bj),
        grid=(ni, nj),
        in_specs=[
            pl.BlockSpec((d, bj), lambda i, j: (0, j)),
            pl.BlockSpec((bj, bi), lambda i, j: (j, i)),
            pl.BlockSpec((d, d), lambda i, j: (0, 0)),
        ],
        out_specs=[
            pl.BlockSpec((d, bi), lambda i, j: (0, i)),
            pl.BlockSpec((1, bi), lambda i, j: (0, i)),
        ],
        out_shape=[
            jax.ShapeDtypeStruct((d, n_pad), jnp.bfloat16),
            jax.ShapeDtypeStruct((1, n_pad), jnp.float32),
        ],
        scratch_shapes=[
            pltpu.VMEM((d, bi), jnp.float32),
            pltpu.VMEM((1, bi), jnp.float32),
        ],
        compiler_params=pltpu.CompilerParams(
            dimension_semantics=("parallel", "arbitrary")),
    )(h0t, adj_mat, u1)

    if ni * bi < n_pad:
        h1t = h1t.at[:, ni * bi:].set(0)

    return pl.pallas_call(
        _layer1_body,
        grid=(ni, nj),
        in_specs=[
            pl.BlockSpec((d, bj), lambda i, j: (0, j)),
            pl.BlockSpec((bj, bi), lambda i, j: (j, i)),
            pl.BlockSpec((d, d), lambda i, j: (0, 0)),
            pl.BlockSpec((1, bi), lambda i, j: (0, i)),
        ],
        out_specs=pl.BlockSpec((bi, d), lambda i, j: (i, 0)),
        out_shape=jax.ShapeDtypeStruct((n, d), jnp.float32),
        scratch_shapes=[pltpu.VMEM((d, bi), jnp.float32)],
        compiler_params=pltpu.CompilerParams(
            dimension_semantics=("parallel", "arbitrary")),
    )(h1t, adj_mat, u2, deg)


# speedup vs baseline: 1.0920x; 1.0920x over previous
import functools

import jax
import jax.numpy as jnp
from jax.experimental import pallas as pl
from jax.experimental.pallas import tpu as pltpu


def _cgn_body(xt_ref, au_ref, al_ref, u0t_ref, b0_ref, us_ref, out_ref,
              ht_scr, acc_ref, deg_ref, degall_ref, *, n, bi, bj, h1_tail):
    l = pl.program_id(0)
    i = pl.program_id(1)
    j = pl.program_id(2)
    nj = pl.num_programs(2)
    bh = bj // 2

    @pl.when((l == 0) & (i == 0))
    def _():
        xt_b = xt_ref[:, pl.ds(j * bj, bj)]
        h0t = jax.lax.dot_general(u0t_ref[...], xt_b, (((1,), (0,)), ((), ())),
                                  preferred_element_type=jnp.float32)
        h0t = h0t + b0_ref[...]
        col = jax.lax.broadcasted_iota(jnp.int32, h0t.shape, 1)
        h0t = jnp.where(col < n - j * bj, h0t, 0.0)
        ht_scr[0, :, pl.ds(j * bj, bj)] = h0t.astype(jnp.bfloat16)

    au = au_ref[...].astype(jnp.bfloat16)
    al = al_ref[...].astype(jnp.bfloat16)
    ht_u = ht_scr[l, :, pl.ds(j * bj, bh)]
    ht_l = ht_scr[l, :, pl.ds(j * bj + bh, bh)]
    dn = (((1,), (0,)), ((), ()))
    part = (jax.lax.dot_general(ht_u, au, dn, preferred_element_type=jnp.float32)
            + jax.lax.dot_general(ht_l, al, dn, preferred_element_type=jnp.float32))

    @pl.when(j == 0)
    def _():
        acc_ref[...] = part

    @pl.when(j != 0)
    def _():
        acc_ref[...] = acc_ref[...] + part

    @pl.when(l == 0)
    def _():
        row = jax.lax.broadcasted_iota(jnp.int32, (1, bh), 1)
        vu = (row < n - j * bj).astype(jnp.bfloat16)
        vl = (row < n - j * bj - bh).astype(jnp.bfloat16)
        degp = (jax.lax.dot_general(vu, au, dn, preferred_element_type=jnp.float32)
                + jax.lax.dot_general(vl, al, dn, preferred_element_type=jnp.float32))

        @pl.when(j == 0)
        def _():
            deg_ref[...] = degp

        @pl.when(j != 0)
        def _():
            deg_ref[...] = deg_ref[...] + degp

        @pl.when(j == nj - 1)
        def _():
            degall_ref[:, pl.ds(i * bi, bi)] = deg_ref[...]

    @pl.when(j == nj - 1)
    def _():
        deg_row = degall_ref[:, pl.ds(i * bi, bi)]
        scaled = (acc_ref[...] / deg_row).astype(jnp.bfloat16)
        y = jax.lax.dot_general(us_ref[0], scaled, dn,
                                preferred_element_type=jnp.float32)
        y = jnp.maximum(y, 0.0)

        @pl.when(l == 0)
        def _():
            col = jax.lax.broadcasted_iota(jnp.int32, y.shape, 1)
            y0 = jnp.where(col < n - i * bi, y, 0.0)
            ht_scr[1, :, pl.ds(i * bi, bi)] = y0.astype(jnp.bfloat16)
            if h1_tail:
                ni_ = pl.num_programs(1)

                @pl.when(i == ni_ - 1)
                def _():
                    d_ = y.shape[0]
                    ht_scr[1, :, pl.ds(ni_ * bi, h1_tail)] = jnp.zeros(
                        (d_, h1_tail), jnp.bfloat16)

        @pl.when(l == 1)
        def _():
            out_ref[...] = jnp.swapaxes(y, 0, 1)

def kernel(x, adj_mat, U0, b0, U1, U2):
    n, d = x.shape
    bi = 1280 if n > 1280 else n // 8
    bj = 2560 if n > 1280 else n // 5
    bh = bj // 2
    ni = -(-n // bi)
    nj = -(-n // bj)
    n_pad = max(ni * bi, nj * bj)

    xt = jnp.zeros((d, n_pad), jnp.bfloat16).at[:, :n].set(
        x.astype(jnp.bfloat16).T)
    u0t = U0.T.astype(jnp.bfloat16)
    us = jnp.stack([U1, U2]).astype(jnp.bfloat16)
    b0c = b0.reshape(d, 1)

    body = functools.partial(_cgn_body, n=n, bi=bi, bj=bj,
                             h1_tail=n_pad - ni * bi)

    return pl.pallas_call(
        body,
        grid=(2, ni, nj),
        in_specs=[
            pl.BlockSpec((d, n_pad), lambda l, i, j: (0, 0)),
            pl.BlockSpec((bh, bi), lambda l, i, j: (2 * j, i)),
            pl.BlockSpec((bh, bi), lambda l, i, j: (2 * j + 1, i)),
            pl.BlockSpec((d, d), lambda l, i, j: (0, 0)),
            pl.BlockSpec((d, 1), lambda l, i, j: (0, 0)),
            pl.BlockSpec((1, d, d), lambda l, i, j: (l, 0, 0)),
        ],
        out_specs=pl.BlockSpec((bi, d), lambda l, i, j: (i, 0)),
        out_shape=jax.ShapeDtypeStruct((n, d), jnp.float32),
        scratch_shapes=[
            pltpu.VMEM((2, d, n_pad), jnp.bfloat16),
            pltpu.VMEM((d, bi), jnp.float32),
            pltpu.VMEM((1, bi), jnp.float32),
            pltpu.VMEM((1, n_pad), jnp.float32),
        ],
    )(xt, adj_mat, adj_mat, u0t, b0c, us)
